# Initial kernel scaffold; baseline (speedup 1.0000x reference)
import jax
import jax.numpy as jnp
from jax.experimental import pallas as pl


def kernel(x, table):
    raise NotImplementedError("write your pallas kernel here")


# speedup vs baseline: 3.6962x; 3.6962x over previous
import functools

import jax
import jax.numpy as jnp
from jax import lax
from jax.experimental import pallas as pl
from jax.experimental.pallas import tpu as pltpu
from jax.experimental.pallas import tpu_sc as plsc

F = 26
B = 4096
D = 64
ROWS_PER_FIELD = 1000
N = B * F
NC = 2
NS = 16
NW = NC * NS
PER_W = N // NW
GSZ = 128
GPW = PER_W // GSZ

_mesh = plsc.VectorSubcoreMesh(core_axis_name="c", subcore_axis_name="s")


@functools.partial(
    pl.kernel,
    out_type=jax.ShapeDtypeStruct((NW, GPW, GSZ, D), jnp.float32),
    mesh=_mesh,
    compiler_params=pltpu.CompilerParams(use_tc_tiling_on_sc=False),
    scratch_types=[
        pltpu.VMEM((GPW, GSZ), jnp.int32),
        pltpu.VMEM((GSZ, D), jnp.float32),
        pltpu.VMEM((GSZ, D), jnp.float32),
        pltpu.SemaphoreType.DMA,
        pltpu.SemaphoreType.DMA,
        pltpu.SemaphoreType.DMA,
        pltpu.SemaphoreType.DMA,
    ],
)
def _emb_lookup(x_hbm, table_hbm, out_hbm, idx_v, buf0, buf1, g0, g1, w0, w1):
    wid = lax.axis_index("s") * NC + lax.axis_index("c")
    pltpu.sync_copy(x_hbm.at[wid], idx_v)

    for g in range(GPW):
        def _add(i, _, g=g):
            sl = pl.ds(i * 16, 16)
            pos = lax.iota(jnp.int32, 16) + (g * GSZ + i * 16)
            idx_v[g, sl] = idx_v[g, sl] + (pos % F) * ROWS_PER_FIELD
            return _
        lax.fori_loop(0, GSZ // 16, _add, 0)

    bufs = (buf0, buf1)
    gsems = (g0, g1)
    wsems = (w0, w1)

    pltpu.async_copy(table_hbm.at[idx_v.at[0]], bufs[0], gsems[0])
    for g in range(GPW):
        p = g & 1
        if g + 1 < GPW:
            if g >= 1:
                pltpu.make_async_copy(bufs[1 - p], out_hbm.at[wid, g - 1],
                                      wsems[1 - p]).wait()
            pltpu.async_copy(table_hbm.at[idx_v.at[g + 1]], bufs[1 - p],
                             gsems[1 - p])
        pltpu.make_async_copy(table_hbm.at[idx_v.at[g]], bufs[p],
                              gsems[p]).wait()
        pltpu.async_copy(bufs[p], out_hbm.at[wid, g], wsems[p])
    pltpu.make_async_copy(bufs[(GPW - 2) & 1], out_hbm.at[wid, GPW - 2],
                          wsems[(GPW - 2) & 1]).wait()
    pltpu.make_async_copy(bufs[(GPW - 1) & 1], out_hbm.at[wid, GPW - 1],
                          wsems[(GPW - 1) & 1]).wait()


def kernel(x, table):
    x3 = x.astype(jnp.int32).reshape(NW, GPW, GSZ)
    out = _emb_lookup(x3, table)
    return out.reshape(B, F, D)


# speedup vs baseline: 3.8392x; 1.0387x over previous
import functools

import jax
import jax.numpy as jnp
from jax import lax
from jax.experimental import pallas as pl
from jax.experimental.pallas import tpu as pltpu
from jax.experimental.pallas import tpu_sc as plsc

F = 26
B = 4096
D = 64
ROWS_PER_FIELD = 1000
N = B * F
NC = 2
NS = 16
NW = NC * NS
PER_W = N // NW
GSZ = 128
GPW = PER_W // GSZ

_mesh = plsc.VectorSubcoreMesh(core_axis_name="c", subcore_axis_name="s")

NBUF = 4


@functools.partial(
    pl.kernel,
    out_type=jax.ShapeDtypeStruct((NW, GPW, GSZ, D), jnp.float32),
    mesh=_mesh,
    compiler_params=pltpu.CompilerParams(use_tc_tiling_on_sc=False),
    scratch_types=(
        [pltpu.VMEM((GPW, GSZ), jnp.int32)]
        + [pltpu.VMEM((GSZ, D), jnp.float32)] * NBUF
        + [pltpu.SemaphoreType.DMA] * (2 * NBUF)
    ),
)
def _emb_lookup(x_hbm, table_hbm, out_hbm, idx_v, *bufs_sems):
    bufs = bufs_sems[:NBUF]
    gsems = bufs_sems[NBUF:2 * NBUF]
    wsems = bufs_sems[2 * NBUF:]
    wid = lax.axis_index("s") * NC + lax.axis_index("c")
    pltpu.sync_copy(x_hbm.at[wid], idx_v)

    def add_offsets(g):
        def _add(i, c, g=g):
            sl = pl.ds(i * 16, 16)
            pos = lax.iota(jnp.int32, 16) + (g * GSZ + i * 16)
            idx_v[g, sl] = idx_v[g, sl] + (pos % F) * ROWS_PER_FIELD
            return c
        lax.fori_loop(0, GSZ // 16, _add, 0)

    for g in range(NBUF):
        add_offsets(g)
        pltpu.async_copy(table_hbm.at[idx_v.at[g]], bufs[g], gsems[g])

    for g in range(GPW):
        p = g % NBUF
        j = g - 1 + NBUF
        if g >= 1 and j < GPW:
            q = (g - 1) % NBUF
            add_offsets(j)
            pltpu.make_async_copy(bufs[q], out_hbm.at[wid, g - 1],
                                  wsems[q]).wait()
            pltpu.async_copy(table_hbm.at[idx_v.at[j]], bufs[q], gsems[q])
        pltpu.make_async_copy(table_hbm.at[idx_v.at[g]], bufs[p],
                              gsems[p]).wait()
        pltpu.async_copy(bufs[p], out_hbm.at[wid, g], wsems[p])

    for g in range(GPW - NBUF, GPW):
        p = g % NBUF
        pltpu.make_async_copy(bufs[p], out_hbm.at[wid, g], wsems[p]).wait()


def kernel(x, table):
    x3 = x.astype(jnp.int32).reshape(NW, GPW, GSZ)
    out = _emb_lookup(x3, table)
    return out.reshape(B, F, D)
